# baseline (speedup 1.0000x reference)
import jax
import jax.numpy as jnp
from jax.experimental import pallas as pl

B = 128
N = 55
D = 3
H = 64
L = 4
NP = 56
E = NP * NP
CRL = 15.0 / L


def _bf16_round(x):
    return x.astype(jnp.bfloat16).astype(jnp.float32)


_LOG2E = 1.4426950408889634
_LN2 = 0.6931471805599453


def _exp_neg(x):
    z = jnp.maximum(x * _LOG2E, -150.0)
    n = jnp.floor(z + 0.5)
    r = (z - n) * _LN2
    p = 1.0 + r * (1.0 + r * (0.5 + r * (
        1.0 / 6 + r * (1.0 / 24 + r * (1.0 / 120 + r * (
            1.0 / 720 + r * (1.0 / 5040)))))))
    return jnp.exp2(n) * p


def _recip(d):
    r = 1.0 / d
    r = r * (2.0 - d * r)
    return r * (2.0 - d * r)


def _sigmoid(x):
    t = _exp_neg(-jnp.abs(x))
    pos = _recip(1.0 + t)
    return jnp.where(x >= 0, pos, 1.0 - pos)


def _silu(x):
    return x * _sigmoid(x)


def _tanh(x):
    t = _exp_neg(-2.0 * jnp.abs(x))
    m = (1.0 - t) * _recip(1.0 + t)
    return jnp.where(x >= 0, m, -m)


def _egnn_kernel(cc_ref, cr_ref, h0_ref, mask_ref, eye_ref, w_ref, w12_ref,
                 wn1_ref, v_ref, out_ref):
    f32 = jnp.float32
    silu = jax.nn.silu

    def dot(a, b):
        return jnp.dot(a, b, preferred_element_type=f32)

    def dot_hi(a, b):
        return jnp.dot(a, b, preferred_element_type=f32,
                       precision=jax.lax.Precision.HIGHEST)

    ccol = [cc_ref[0][:, d:d + 1] for d in range(D)]
    crow = [cr_ref[0][d:d + 1, :] for d in range(D)]
    ccol0 = list(ccol)
    h = jnp.broadcast_to(h0_ref[0:1, :], (NP, H))

    emask = mask_ref[:, :]
    eye = eye_ref[:, :]
    ones = eye_ref[0:1, :] * 0.0 + 1.0
    rmask = (jax.lax.broadcasted_iota(jnp.int32, (NP, 1), 0) < N).astype(f32)

    for l in range(L):
        We2 = w_ref[l, 2]
        Wc1 = w_ref[l, 3]
        Wn2 = w_ref[l, 6]
        wa_col = w_ref[l, 7][:, 0:1]
        wc2_col = w_ref[l, 7][:, 1:2]
        wr = v_ref[l, 0:1, :]
        be1 = v_ref[l, 1:2, :]
        be2 = v_ref[l, 2:3, :]
        bc1 = v_ref[l, 3:4, :]
        bn1 = v_ref[l, 4:5, :]
        bn2 = v_ref[l, 5:6, :]
        ba = v_ref[l, 8:9, 0:1]

        dc = [ccol[d] - crow[d] for d in range(D)]
        radial = dc[0] * dc[0] + dc[1] * dc[1] + dc[2] * dc[2]

        hi = jnp.broadcast_to(h[:, None, :], (NP, NP, H)).reshape(E, H)
        hj = jnp.broadcast_to(h[None, :, :], (NP, NP, H)).reshape(E, H)
        hh = jnp.concatenate([hi, hj], axis=1)
        rad_bf = _bf16_round(radial)
        wr_bf = _bf16_round(wr)
        pre = (dot(_bf16_round(hh), w12_ref[l]).reshape(NP, NP, H)
               + rad_bf[:, :, None] * wr_bf.reshape(1, 1, H))
        m1 = silu(pre).reshape(E, H)
        m2 = silu(dot(_bf16_round(m1), We2) + be2)
        gate = jax.nn.sigmoid(dot(_bf16_round(m2), wa_col) + ba) * emask
        m = m2 * gate

        t = silu(dot(_bf16_round(m), Wc1) + bc1)
        phi = jnp.tanh(dot(_bf16_round(t), wc2_col)) * (CRL * emask)
        phi56 = phi.reshape(NP, NP)

        for d in range(D):
            cu = dot_hi(dc[d] * phi56, ones.reshape(NP, 1))
            ccol[d] = ccol[d] + cu
            crow[d] = crow[d] + jax.lax.dot_general(
                cu, eye, (((0,), (0,)), ((), ())),
                preferred_element_type=f32,
                precision=jax.lax.Precision.HIGHEST)

        m3 = m.reshape(NP, NP, H)
        m_agg = m3[:, 0, :]
        for j in range(1, NP):
            m_agg = m_agg + m3[:, j, :]

        cat = jnp.concatenate([h, m_agg], axis=1)
        hp = silu(dot(_bf16_round(cat), wn1_ref[l]) + bn1)
        h = h + dot(_bf16_round(hp), Wn2) + bn2

    inv_n = 1.0 / N
    for d in range(D):
        vel = ccol[d] - ccol0[d]
        mean = jnp.sum(vel * rmask, axis=0, keepdims=True) * inv_n
        out_ref[0, :, d:d + 1] = vel - mean


@jax.jit
def kernel(x, params):
    coords = x.reshape(B, N, D)
    ccol = jnp.pad(coords, ((0, 0), (0, NP - N), (0, 8 - D)))
    crow = jnp.pad(coords.transpose(0, 2, 1),
                   ((0, 0), (0, 8 - D), (0, NP - N)))

    h0 = (params['emb_w'] + params['emb_b'][None, :]).reshape(1, H)
    h0 = jnp.pad(h0, ((0, 7), (0, 0)))

    ii, jj = jnp.mgrid[0:NP, 0:NP]
    emask = ((ii != jj) & (jj < N)).astype(jnp.float32).reshape(E, 1)
    eye = jnp.eye(NP, dtype=jnp.float32)

    Ws, Vs = [], []
    for p in params['layers']:
        We1 = p['We1']
        Wn1 = p['Wn1']
        small = jnp.zeros((H, H), jnp.float32)
        small = small.at[:, 0:1].set(p['Wa']).at[:, 1:2].set(p['Wc2'])
        Ws.append(jnp.stack([
            We1[:H], We1[H:2 * H], p['We2'], p['Wc1'],
            Wn1[:H], Wn1[H:], p['Wn2'], small,
        ]))
        vec = jnp.stack([
            We1[2 * H], p['be1'], p['be2'], p['bc1'], p['bn1'], p['bn2'],
            p['Wa'][:, 0], p['Wc2'][:, 0],
            jnp.broadcast_to(p['ba'], (H,)),
        ])
        Vs.append(jnp.pad(vec, ((0, 7), (0, 0))))
    _rt = lambda a: a.astype(jnp.bfloat16).astype(jnp.float32)
    Wstk = _rt(jnp.stack(Ws))
    Vstk = jnp.stack(Vs)
    W12stk = _rt(jnp.stack([p['We1'][:2 * H] for p in params['layers']]))
    Wn1stk = _rt(jnp.stack([p['Wn1'] for p in params['layers']]))

    grid = (B,)
    out = pl.pallas_call(
        _egnn_kernel,
        grid=grid,
        in_specs=[
            pl.BlockSpec((1, NP, 8), lambda b: (b, 0, 0)),
            pl.BlockSpec((1, 8, NP), lambda b: (b, 0, 0)),
            pl.BlockSpec((8, H), lambda b: (0, 0)),
            pl.BlockSpec((E, 1), lambda b: (0, 0)),
            pl.BlockSpec((NP, NP), lambda b: (0, 0)),
            pl.BlockSpec((L, 8, H, H), lambda b: (0, 0, 0, 0)),
            pl.BlockSpec((L, 2 * H, H), lambda b: (0, 0, 0)),
            pl.BlockSpec((L, 2 * H, H), lambda b: (0, 0, 0)),
            pl.BlockSpec((L, 16, H), lambda b: (0, 0, 0)),
        ],
        out_specs=pl.BlockSpec((1, NP, 8), lambda b: (b, 0, 0)),
        out_shape=jax.ShapeDtypeStruct((B, NP, 8), jnp.float32),
    )(ccol, crow, h0, emask, eye, Wstk, W12stk, Wn1stk, Vstk)

    vel = out[:, :N, :D].reshape(B, N * D)
    return vel * params['output_scale']
